# baseline (speedup 1.0000x reference)
import functools

import jax
import jax.numpy as jnp
from jax import lax
from jax.experimental import pallas as pl
from jax.experimental.pallas import tpu as pltpu
from jax.experimental.pallas import tpu_sc as plsc

DIM = 64
LANES = 16
CHUNK = 128
MARGIN = 1.0
REL_ROWS = 1001
REL_WORDS = REL_ROWS * DIM
CW = CHUNK * DIM


def _build(n_rows):
    info = plsc.get_sparse_core_info()
    nc, ns = info.num_cores, info.num_subcores
    n_workers = nc * ns
    rows_per_w = n_rows // n_workers
    n_chunks = rows_per_w // CHUNK
    n_groups = CHUNK // LANES
    mesh = plsc.VectorSubcoreMesh(core_axis_name="c", subcore_axis_name="s")

    @functools.partial(
        pl.kernel,
        mesh=mesh,
        compiler_params=pltpu.CompilerParams(needs_layout_passes=False),
        out_type=(
            jax.ShapeDtypeStruct((n_rows,), jnp.float32),
            jax.ShapeDtypeStruct((n_rows * DIM,), jnp.float32),
        ),
        scratch_types=[
            pltpu.VMEM((2, CHUNK), jnp.int32),
            pltpu.VMEM((2, CHUNK), jnp.int32),
            pltpu.VMEM((2, CHUNK), jnp.int32),
            pltpu.VMEM((2, CHUNK), jnp.int32),
            pltpu.VMEM((2, CHUNK), jnp.float32),
            pltpu.VMEM((CW,), jnp.float32),
            pltpu.VMEM((CW,), jnp.float32),
            pltpu.VMEM((CW,), jnp.float32),
            pltpu.VMEM((CW,), jnp.float32),
            pltpu.VMEM((CW,), jnp.float32),
            pltpu.VMEM((REL_WORDS,), jnp.float32),
            pltpu.VMEM((CHUNK,), jnp.float32),
            pltpu.VMEM((LANES * LANES,), jnp.float32),
            pltpu.SemaphoreType.DMA,
            pltpu.SemaphoreType.DMA,
            pltpu.SemaphoreType.DMA,
            pltpu.SemaphoreType.DMA,
            pltpu.SemaphoreType.DMA,
        ],
    )
    def k(ph_hbm, nh_hbm, pti_hbm, nti_hbm, pri_hbm, nri_hbm, mask_hbm,
          relf_hbm, tailf_hbm, loss_hbm, relout_hbm,
          pti2, nti2, pri2, nri2, mask2, ph_v, nh_v, pt_v, nt_v, pr_v,
          rel_v, loss_v, stage_v, sem_in, sem_heads, sem_rows, sem_out,
          sem_tbl):
        wid = lax.axis_index("s") * nc + lax.axis_index("c")
        base0 = wid * rows_per_w
        iota = lax.iota(jnp.int32, LANES)

        def fire_idx(ci, b):
            sl = pl.ds(base0 + ci * CHUNK, CHUNK)
            pltpu.async_copy(pti_hbm.at[sl], pti2.at[b], sem_in)
            pltpu.async_copy(nti_hbm.at[sl], nti2.at[b], sem_in)
            pltpu.async_copy(pri_hbm.at[sl], pri2.at[b], sem_in)
            pltpu.async_copy(nri_hbm.at[sl], nri2.at[b], sem_in)
            pltpu.async_copy(mask_hbm.at[sl], mask2.at[b], sem_in)

        def drain_idx():
            for _ in range(4):
                pltpu.make_async_copy(pti_hbm.at[pl.ds(0, CHUNK)],
                                      pti2.at[0], sem_in).wait()
            pltpu.make_async_copy(mask_hbm.at[pl.ds(0, CHUNK)],
                                  mask2.at[0], sem_in).wait()

        def fire_heads(ci):
            sl = pl.ds((base0 + ci * CHUNK) * DIM, CW)
            pltpu.async_copy(ph_hbm.at[sl], ph_v, sem_heads)
            pltpu.async_copy(nh_hbm.at[sl], nh_v, sem_heads)

        def drain_heads():
            pltpu.make_async_copy(ph_hbm.at[pl.ds(0, CW)],
                                  ph_v, sem_heads).wait()
            pltpu.make_async_copy(ph_hbm.at[pl.ds(0, CW)],
                                  nh_v, sem_heads).wait()

        def fire_group(b, g):
            gsl = pl.ds(g * LANES, LANES)
            ptv = pti2[b, gsl]
            ntv = nti2[b, gsl]
            prv = pri2[b, gsl]
            for rr in range(LANES):
                r = g * LANES + rr
                pltpu.async_copy(tailf_hbm.at[pl.ds(ptv[rr] * DIM, DIM)],
                                 pt_v.at[pl.ds(r * DIM, DIM)], sem_rows)
                pltpu.async_copy(tailf_hbm.at[pl.ds(ntv[rr] * DIM, DIM)],
                                 nt_v.at[pl.ds(r * DIM, DIM)], sem_rows)
                pltpu.async_copy(relf_hbm.at[pl.ds(prv[rr] * DIM, DIM)],
                                 pr_v.at[pl.ds(r * DIM, DIM)], sem_rows)

        def drain_rows_quarter():
            def d(i, c):
                pltpu.make_async_copy(tailf_hbm.at[pl.ds(0, DIM)],
                                      pt_v.at[pl.ds(0, DIM)],
                                      sem_rows).wait()
                return c

            lax.fori_loop(0, 2 * 3 * LANES, d, 0)

        def drain_outs():
            pltpu.make_async_copy(loss_hbm.at[pl.ds(0, CHUNK)],
                                  loss_v, sem_out).wait()
            pltpu.make_async_copy(relout_hbm.at[pl.ds(0, CW)],
                                  pr_v, sem_out).wait()

        def chunk_body(ci, carry):
            b = lax.rem(ci, 2)
            sl = pl.ds(base0 + ci * CHUNK, CHUNK)
            sl64 = pl.ds((base0 + ci * CHUNK) * DIM, CW)

            @pl.when(ci > 0)
            def _():
                drain_outs()

            drain_idx()
            fire_heads(ci)

            @pl.when(ci + 1 < n_chunks)
            def _():
                fire_idx(ci + 1, 1 - b)

            def fire4(g, c):
                fire_group(b, g)
                return c

            lax.fori_loop(0, n_groups // 2, fire4, 0)
            drain_heads()

            def grand(gg, c):
                @pl.when(lax.rem(gg, 2) == 0)
                def _():
                    drain_rows_quarter()

                @pl.when(gg < n_groups // 2)
                def _():
                    fire_group(b, gg + n_groups // 2)

                gsl = pl.ds(gg * LANES, LANES)
                nrvec = nri2[b, gsl] * DIM
                for rr in range(LANES):
                    r = gg * LANES + rr
                    nbase = nrvec[rr]
                    acc0 = jnp.zeros((LANES,), jnp.float32)
                    acc1 = jnp.zeros((LANES,), jnp.float32)
                    for j in range(DIM // LANES):
                        js = pl.ds(r * DIM + j * LANES, LANES)
                        nrj = plsc.load_gather(
                            rel_v, [nbase + j * LANES + iota])
                        pterm = jnp.abs(ph_v[js] + pr_v[js] - pt_v[js])
                        nterm = jnp.abs(nh_v[js] + nrj - nt_v[js])
                        if j % 2 == 0:
                            acc0 = acc0 + (pterm - nterm)
                        else:
                            acc1 = acc1 + (pterm - nterm)
                    stage_v[pl.ds(rr * LANES, LANES)] = acc0 + acc1
                tot = jnp.zeros((LANES,), jnp.float32)
                for j in range(LANES):
                    tot = tot + plsc.load_gather(stage_v, [iota * LANES + j])
                loss_v[gsl] = jnp.maximum(mask2[b, gsl] * tot + MARGIN, 0.0)
                return c

            lax.fori_loop(0, n_groups, grand, 0)
            pltpu.async_copy(loss_v, loss_hbm.at[sl], sem_out)
            pltpu.async_copy(pr_v, relout_hbm.at[sl64], sem_out)
            return carry

        pltpu.async_copy(relf_hbm, rel_v, sem_tbl)
        fire_idx(0, 0)
        pltpu.make_async_copy(relf_hbm, rel_v, sem_tbl).wait()
        lax.fori_loop(0, n_chunks, chunk_body, 0)
        drain_outs()

    return k


def kernel(positive_head, positive_tail, positive_relation, negtive_head,
           negtive_tail, negtive_relation, attn_mask, rel_table, tail_table):
    b, l, d = positive_head.shape
    n = b * l
    ph = positive_head.reshape(n * d)
    nh = negtive_head.reshape(n * d)
    pti = positive_tail.reshape(n).astype(jnp.int32)
    nti = negtive_tail.reshape(n).astype(jnp.int32)
    pri = positive_relation.reshape(n).astype(jnp.int32)
    nri = negtive_relation.reshape(n).astype(jnp.int32)
    mask = attn_mask.reshape(n).astype(jnp.float32)
    rel_flat = rel_table.reshape(-1)
    tail_flat = tail_table.reshape(-1)
    loss, rel_rows = _build(n)(
        ph, nh, pti, nti, pri, nri, mask, rel_flat, tail_flat)
    return loss.reshape(n, 1), rel_rows.reshape(b, l, d)


# speedup vs baseline: 1.2414x; 1.2414x over previous
import functools

import jax
import jax.numpy as jnp
from jax import lax
from jax.experimental import pallas as pl
from jax.experimental.pallas import tpu as pltpu
from jax.experimental.pallas import tpu_sc as plsc

DIM = 64
LANES = 16
CHUNK = 64
MARGIN = 1.0
REL_ROWS = 1001
REL_WORDS = REL_ROWS * DIM


def _build(n_rows):
    info = plsc.get_sparse_core_info()
    nc, ns = info.num_cores, info.num_subcores
    n_workers = nc * ns
    rows_per_w = n_rows // n_workers
    n_chunks = rows_per_w // CHUNK
    n_groups = CHUNK // LANES
    mesh = plsc.VectorSubcoreMesh(core_axis_name="c", subcore_axis_name="s")

    @functools.partial(
        pl.kernel,
        mesh=mesh,
        compiler_params=pltpu.CompilerParams(needs_layout_passes=False),
        out_type=(
            jax.ShapeDtypeStruct((n_rows,), jnp.float32),
            jax.ShapeDtypeStruct((n_rows, DIM), jnp.float32),
        ),
        scratch_types=[
            pltpu.VMEM((2, CHUNK), jnp.int32),
            pltpu.VMEM((2, CHUNK), jnp.int32),
            pltpu.VMEM((2, CHUNK), jnp.int32),
            pltpu.VMEM((2, CHUNK), jnp.int32),
            pltpu.VMEM((2, CHUNK), jnp.float32),
            pltpu.VMEM((CHUNK, DIM), jnp.float32),
            pltpu.VMEM((CHUNK, DIM), jnp.float32),
            pltpu.VMEM((CHUNK, DIM), jnp.float32),
            pltpu.VMEM((CHUNK, DIM), jnp.float32),
            pltpu.VMEM((CHUNK, DIM), jnp.float32),
            pltpu.VMEM((REL_WORDS,), jnp.float32),
            pltpu.VMEM((CHUNK,), jnp.float32),
            pltpu.VMEM((LANES * LANES,), jnp.float32),
            pltpu.SemaphoreType.DMA,
            pltpu.SemaphoreType.DMA,
            pltpu.SemaphoreType.DMA,
            pltpu.SemaphoreType.DMA,
            pltpu.SemaphoreType.DMA,
        ],
    )
    def k(ph_hbm, nh_hbm, pti_hbm, nti_hbm, pri_hbm, nri_hbm, mask_hbm,
          relf_hbm, tail_hbm, loss_hbm, relout_hbm,
          pti2, nti2, pri2, nri2, mask2, ph_v, nh_v, pt_v, nt_v, pr_v,
          rel_v, loss_v, stage_v, sem_in, sem_heads, sem_rows, sem_out,
          sem_tbl):
        wid = lax.axis_index("s") * nc + lax.axis_index("c")
        base0 = wid * rows_per_w
        iota = lax.iota(jnp.int32, LANES)

        def fire_idx(ci, b):
            sl = pl.ds(base0 + ci * CHUNK, CHUNK)
            pltpu.async_copy(pti_hbm.at[sl], pti2.at[b], sem_in)
            pltpu.async_copy(nti_hbm.at[sl], nti2.at[b], sem_in)
            pltpu.async_copy(pri_hbm.at[sl], pri2.at[b], sem_in)
            pltpu.async_copy(nri_hbm.at[sl], nri2.at[b], sem_in)
            pltpu.async_copy(mask_hbm.at[sl], mask2.at[b], sem_in)

        def drain_idx():
            for _ in range(4):
                pltpu.make_async_copy(pti_hbm.at[pl.ds(0, CHUNK)],
                                      pti2.at[0], sem_in).wait()
            pltpu.make_async_copy(mask_hbm.at[pl.ds(0, CHUNK)],
                                  mask2.at[0], sem_in).wait()

        def fire_heads(ci):
            sl = pl.ds(base0 + ci * CHUNK, CHUNK)
            pltpu.async_copy(ph_hbm.at[sl], ph_v, sem_heads)
            pltpu.async_copy(nh_hbm.at[sl], nh_v, sem_heads)

        def drain_heads():
            pltpu.make_async_copy(ph_hbm.at[pl.ds(0, CHUNK)],
                                  ph_v, sem_heads).wait()
            pltpu.make_async_copy(ph_hbm.at[pl.ds(0, CHUNK)],
                                  nh_v, sem_heads).wait()

        def fire_group(b, g):
            gsl = pl.ds(g * LANES, LANES)
            ptv = pti2[b, gsl]
            ntv = nti2[b, gsl]
            for rr in range(LANES):
                r = g * LANES + rr
                pltpu.async_copy(tail_hbm.at[pl.ds(ptv[rr], 1)],
                                 pt_v.at[pl.ds(r, 1)], sem_rows)
                pltpu.async_copy(tail_hbm.at[pl.ds(ntv[rr], 1)],
                                 nt_v.at[pl.ds(r, 1)], sem_rows)

        def drain_rows_quarter():
            def d(i, c):
                pltpu.make_async_copy(tail_hbm.at[pl.ds(0, 1)],
                                      pt_v.at[pl.ds(0, 1)], sem_rows).wait()
                return c

            lax.fori_loop(0, 2 * 2 * LANES, d, 0)

        def drain_outs():
            pltpu.make_async_copy(loss_hbm.at[pl.ds(0, CHUNK)],
                                  loss_v, sem_out).wait()
            pltpu.make_async_copy(relout_hbm.at[pl.ds(0, CHUNK)],
                                  pr_v, sem_out).wait()

        def chunk_body(ci, carry):
            b = lax.rem(ci, 2)
            sl = pl.ds(base0 + ci * CHUNK, CHUNK)

            @pl.when(ci > 0)
            def _():
                drain_outs()

            drain_idx()
            fire_heads(ci)

            @pl.when(ci + 1 < n_chunks)
            def _():
                fire_idx(ci + 1, 1 - b)

            def fire4(g, c):
                fire_group(b, g)
                return c

            lax.fori_loop(0, n_groups // 2, fire4, 0)
            drain_heads()

            def grand(gg, c):
                @pl.when(lax.rem(gg, 2) == 0)
                def _():
                    drain_rows_quarter()

                @pl.when(gg < n_groups // 2)
                def _():
                    fire_group(b, gg + n_groups // 2)

                gsl = pl.ds(gg * LANES, LANES)
                prvec = pri2[b, gsl] * DIM
                nrvec = nri2[b, gsl] * DIM
                for rr in range(LANES):
                    r = gg * LANES + rr
                    pbase = prvec[rr]
                    nbase = nrvec[rr]
                    acc0 = jnp.zeros((LANES,), jnp.float32)
                    acc1 = jnp.zeros((LANES,), jnp.float32)
                    for j in range(DIM // LANES):
                        js = pl.ds(j * LANES, LANES)
                        prj = plsc.load_gather(
                            rel_v, [pbase + j * LANES + iota])
                        nrj = plsc.load_gather(
                            rel_v, [nbase + j * LANES + iota])
                        pr_v[r, js] = prj
                        pterm = jnp.abs(ph_v[r, js] + prj - pt_v[r, js])
                        nterm = jnp.abs(nh_v[r, js] + nrj - nt_v[r, js])
                        if j % 2 == 0:
                            acc0 = acc0 + (pterm - nterm)
                        else:
                            acc1 = acc1 + (pterm - nterm)
                    stage_v[pl.ds(rr * LANES, LANES)] = acc0 + acc1
                tot = jnp.zeros((LANES,), jnp.float32)
                for j in range(LANES):
                    tot = tot + plsc.load_gather(stage_v, [iota * LANES + j])
                loss_v[gsl] = jnp.maximum(mask2[b, gsl] * tot + MARGIN, 0.0)
                return c

            lax.fori_loop(0, n_groups, grand, 0)
            pltpu.async_copy(loss_v, loss_hbm.at[sl], sem_out)
            pltpu.async_copy(pr_v, relout_hbm.at[sl], sem_out)
            return carry

        pltpu.async_copy(relf_hbm, rel_v, sem_tbl)
        fire_idx(0, 0)
        pltpu.make_async_copy(relf_hbm, rel_v, sem_tbl).wait()
        lax.fori_loop(0, n_chunks, chunk_body, 0)
        drain_outs()

    return k


def kernel(positive_head, positive_tail, positive_relation, negtive_head,
           negtive_tail, negtive_relation, attn_mask, rel_table, tail_table):
    b, l, d = positive_head.shape
    n = b * l
    ph = positive_head.reshape(n, d)
    nh = negtive_head.reshape(n, d)
    pti = positive_tail.reshape(n).astype(jnp.int32)
    nti = negtive_tail.reshape(n).astype(jnp.int32)
    pri = positive_relation.reshape(n).astype(jnp.int32)
    nri = negtive_relation.reshape(n).astype(jnp.int32)
    mask = attn_mask.reshape(n).astype(jnp.float32)
    rel_flat = rel_table.reshape(-1)
    loss, rel_rows = _build(n)(
        ph, nh, pti, nti, pri, nri, mask, rel_flat, tail_table)
    return loss.reshape(n, 1), rel_rows.reshape(b, l, d)


# speedup vs baseline: 1.2950x; 1.0431x over previous
import functools

import jax
import jax.numpy as jnp
from jax import lax
from jax.experimental import pallas as pl
from jax.experimental.pallas import tpu as pltpu
from jax.experimental.pallas import tpu_sc as plsc

DIM = 64
LANES = 16
CHUNK = 64
MARGIN = 1.0
REL_ROWS = 1001
REL_WORDS = REL_ROWS * DIM


def _build(n_rows):
    info = plsc.get_sparse_core_info()
    nc, ns = info.num_cores, info.num_subcores
    n_workers = nc * ns
    rows_per_w = n_rows // n_workers
    n_chunks = rows_per_w // CHUNK
    n_groups = CHUNK // LANES
    mesh = plsc.VectorSubcoreMesh(core_axis_name="c", subcore_axis_name="s")

    @functools.partial(
        pl.kernel,
        mesh=mesh,
        compiler_params=pltpu.CompilerParams(needs_layout_passes=False),
        out_type=(
            jax.ShapeDtypeStruct((n_rows,), jnp.float32),
            jax.ShapeDtypeStruct((n_rows, DIM), jnp.float32),
        ),
        scratch_types=[
            pltpu.VMEM((2, CHUNK), jnp.int32),
            pltpu.VMEM((2, CHUNK), jnp.int32),
            pltpu.VMEM((2, CHUNK), jnp.int32),
            pltpu.VMEM((2, CHUNK), jnp.int32),
            pltpu.VMEM((2, CHUNK), jnp.float32),
            pltpu.VMEM((CHUNK, DIM), jnp.float32),
            pltpu.VMEM((CHUNK, DIM), jnp.float32),
            pltpu.VMEM((2, CHUNK, DIM), jnp.float32),
            pltpu.VMEM((2, CHUNK, DIM), jnp.float32),
            pltpu.VMEM((CHUNK, DIM), jnp.float32),
            pltpu.VMEM((REL_WORDS,), jnp.float32),
            pltpu.VMEM((CHUNK,), jnp.float32),
            pltpu.VMEM((LANES * LANES,), jnp.float32),
            pltpu.SemaphoreType.DMA,
            pltpu.SemaphoreType.DMA,
            pltpu.SemaphoreType.DMA,
            pltpu.SemaphoreType.DMA,
            pltpu.SemaphoreType.DMA,
        ],
    )
    def k(ph_hbm, nh_hbm, pti_hbm, nti_hbm, pri_hbm, nri_hbm, mask_hbm,
          relf_hbm, tail_hbm, loss_hbm, relout_hbm,
          pti2, nti2, pri2, nri2, mask2, ph_v, nh_v, pt_v, nt_v, pr_v,
          rel_v, loss_v, stage_v, sem_in, sem_heads, sem_rows, sem_out,
          sem_tbl):
        wid = lax.axis_index("s") * nc + lax.axis_index("c")
        base0 = wid * rows_per_w
        iota = lax.iota(jnp.int32, LANES)

        def fire_idx(ci, b):
            sl = pl.ds(base0 + ci * CHUNK, CHUNK)
            pltpu.async_copy(pti_hbm.at[sl], pti2.at[b], sem_in)
            pltpu.async_copy(nti_hbm.at[sl], nti2.at[b], sem_in)
            pltpu.async_copy(pri_hbm.at[sl], pri2.at[b], sem_in)
            pltpu.async_copy(nri_hbm.at[sl], nri2.at[b], sem_in)
            pltpu.async_copy(mask_hbm.at[sl], mask2.at[b], sem_in)

        def drain_idx():
            for _ in range(4):
                pltpu.make_async_copy(pti_hbm.at[pl.ds(0, CHUNK)],
                                      pti2.at[0], sem_in).wait()
            pltpu.make_async_copy(mask_hbm.at[pl.ds(0, CHUNK)],
                                  mask2.at[0], sem_in).wait()

        def fire_heads(ci):
            sl = pl.ds(base0 + ci * CHUNK, CHUNK)
            pltpu.async_copy(ph_hbm.at[sl], ph_v, sem_heads)
            pltpu.async_copy(nh_hbm.at[sl], nh_v, sem_heads)

        def drain_heads():
            pltpu.make_async_copy(ph_hbm.at[pl.ds(0, CHUNK)],
                                  ph_v, sem_heads).wait()
            pltpu.make_async_copy(ph_hbm.at[pl.ds(0, CHUNK)],
                                  nh_v, sem_heads).wait()

        def fire_group(b, g):
            gsl = pl.ds(g * LANES, LANES)
            ptv = pti2[b, gsl]
            ntv = nti2[b, gsl]
            for rr in range(LANES):
                r = g * LANES + rr
                pltpu.async_copy(tail_hbm.at[pl.ds(ptv[rr], 1)],
                                 pt_v.at[b, pl.ds(r, 1)], sem_rows)
                pltpu.async_copy(tail_hbm.at[pl.ds(ntv[rr], 1)],
                                 nt_v.at[b, pl.ds(r, 1)], sem_rows)

        def drain_rows_all():
            def d(i, c):
                pltpu.make_async_copy(tail_hbm.at[pl.ds(0, 1)],
                                      pt_v.at[0, pl.ds(0, 1)],
                                      sem_rows).wait()
                return c

            lax.fori_loop(0, 2 * CHUNK, d, 0)

        def drain_outs():
            pltpu.make_async_copy(loss_hbm.at[pl.ds(0, CHUNK)],
                                  loss_v, sem_out).wait()
            pltpu.make_async_copy(relout_hbm.at[pl.ds(0, CHUNK)],
                                  pr_v, sem_out).wait()

        def chunk_body(ci, carry):
            b = lax.rem(ci, 2)
            sl = pl.ds(base0 + ci * CHUNK, CHUNK)

            @pl.when(ci > 0)
            def _():
                drain_outs()

            fire_heads(ci)
            drain_rows_all()

            @pl.when(ci + 1 < n_chunks)
            def _():
                drain_idx()

            drain_heads()

            def grand(gg, c):
                @pl.when(ci + 1 < n_chunks)
                def _():
                    fire_group(1 - b, gg)

                gsl = pl.ds(gg * LANES, LANES)
                prvec = pri2[b, gsl] * DIM
                nrvec = nri2[b, gsl] * DIM
                for rr in range(LANES):
                    r = gg * LANES + rr
                    pbase = prvec[rr]
                    nbase = nrvec[rr]
                    acc0 = jnp.zeros((LANES,), jnp.float32)
                    acc1 = jnp.zeros((LANES,), jnp.float32)
                    for j in range(DIM // LANES):
                        js = pl.ds(j * LANES, LANES)
                        prj = plsc.load_gather(
                            rel_v, [pbase + j * LANES + iota])
                        nrj = plsc.load_gather(
                            rel_v, [nbase + j * LANES + iota])
                        pr_v[r, js] = prj
                        pterm = jnp.abs(ph_v[r, js] + prj - pt_v[b, r, js])
                        nterm = jnp.abs(nh_v[r, js] + nrj - nt_v[b, r, js])
                        if j % 2 == 0:
                            acc0 = acc0 + (pterm - nterm)
                        else:
                            acc1 = acc1 + (pterm - nterm)
                    stage_v[pl.ds(rr * LANES, LANES)] = acc0 + acc1
                tot = jnp.zeros((LANES,), jnp.float32)
                for j in range(LANES):
                    tot = tot + plsc.load_gather(stage_v, [iota * LANES + j])
                loss_v[gsl] = jnp.maximum(mask2[b, gsl] * tot + MARGIN, 0.0)
                return c

            lax.fori_loop(0, n_groups, grand, 0)
            pltpu.async_copy(loss_v, loss_hbm.at[sl], sem_out)
            pltpu.async_copy(pr_v, relout_hbm.at[sl], sem_out)

            @pl.when(ci + 2 < n_chunks)
            def _():
                fire_idx(ci + 2, b)

            return carry

        pltpu.async_copy(relf_hbm, rel_v, sem_tbl)
        fire_idx(0, 0)
        pltpu.make_async_copy(relf_hbm, rel_v, sem_tbl).wait()
        drain_idx()

        def fire0(g, c):
            fire_group(0, g)
            return c

        lax.fori_loop(0, n_groups, fire0, 0)
        fire_idx(1, 1)
        lax.fori_loop(0, n_chunks, chunk_body, 0)
        drain_outs()

    return k


def kernel(positive_head, positive_tail, positive_relation, negtive_head,
           negtive_tail, negtive_relation, attn_mask, rel_table, tail_table):
    b, l, d = positive_head.shape
    n = b * l
    ph = positive_head.reshape(n, d)
    nh = negtive_head.reshape(n, d)
    pti = positive_tail.reshape(n).astype(jnp.int32)
    nti = negtive_tail.reshape(n).astype(jnp.int32)
    pri = positive_relation.reshape(n).astype(jnp.int32)
    nri = negtive_relation.reshape(n).astype(jnp.int32)
    mask = attn_mask.reshape(n).astype(jnp.float32)
    rel_flat = rel_table.reshape(-1)
    loss, rel_rows = _build(n)(
        ph, nh, pti, nti, pri, nri, mask, rel_flat, tail_table)
    return loss.reshape(n, 1), rel_rows.reshape(b, l, d)


# speedup vs baseline: 1.3080x; 1.0100x over previous
import functools

import jax
import jax.numpy as jnp
from jax import lax
from jax.experimental import pallas as pl
from jax.experimental.pallas import tpu as pltpu
from jax.experimental.pallas import tpu_sc as plsc

DIM = 64
LANES = 16
CHUNK = 64
MARGIN = 1.0
REL_ROWS = 1001
REL_WORDS = REL_ROWS * DIM


def _build(n_rows):
    info = plsc.get_sparse_core_info()
    nc, ns = info.num_cores, info.num_subcores
    n_workers = nc * ns
    rows_per_w = n_rows // n_workers
    n_chunks = rows_per_w // CHUNK
    n_groups = CHUNK // LANES
    mesh = plsc.VectorSubcoreMesh(core_axis_name="c", subcore_axis_name="s")

    @functools.partial(
        pl.kernel,
        mesh=mesh,
        compiler_params=pltpu.CompilerParams(needs_layout_passes=False),
        out_type=(
            jax.ShapeDtypeStruct((n_rows,), jnp.float32),
            jax.ShapeDtypeStruct((n_rows, DIM), jnp.float32),
        ),
        scratch_types=[
            pltpu.VMEM((2, CHUNK), jnp.int32),
            pltpu.VMEM((2, CHUNK), jnp.int32),
            pltpu.VMEM((2, CHUNK), jnp.int32),
            pltpu.VMEM((2, CHUNK), jnp.int32),
            pltpu.VMEM((2, CHUNK), jnp.float32),
            pltpu.VMEM((CHUNK, DIM), jnp.float32),
            pltpu.VMEM((CHUNK, DIM), jnp.float32),
            pltpu.VMEM((2, CHUNK, DIM), jnp.float32),
            pltpu.VMEM((2, CHUNK, DIM), jnp.float32),
            pltpu.VMEM((CHUNK, DIM), jnp.float32),
            pltpu.VMEM((REL_WORDS,), jnp.float32),
            pltpu.VMEM((rows_per_w,), jnp.float32),
            pltpu.VMEM((LANES * LANES,), jnp.float32),
            pltpu.SemaphoreType.DMA,
            pltpu.SemaphoreType.DMA,
            pltpu.SemaphoreType.DMA,
            pltpu.SemaphoreType.DMA,
            pltpu.SemaphoreType.DMA,
        ],
    )
    def k(ph_hbm, nh_hbm, pti_hbm, nti_hbm, pri_hbm, nri_hbm, mask_hbm,
          relf_hbm, tail_hbm, loss_hbm, relout_hbm,
          pti2, nti2, pri2, nri2, mask2, ph_v, nh_v, pt_v, nt_v, pr_v,
          rel_v, loss_v, stage_v, sem_in, sem_heads, sem_rows, sem_out,
          sem_tbl):
        wid = lax.axis_index("s") * nc + lax.axis_index("c")
        base0 = wid * rows_per_w
        iota = lax.iota(jnp.int32, LANES)

        def fire_idx(ci, b):
            sl = pl.ds(base0 + ci * CHUNK, CHUNK)
            pltpu.async_copy(pti_hbm.at[sl], pti2.at[b], sem_in)
            pltpu.async_copy(nti_hbm.at[sl], nti2.at[b], sem_in)
            pltpu.async_copy(pri_hbm.at[sl], pri2.at[b], sem_in)
            pltpu.async_copy(nri_hbm.at[sl], nri2.at[b], sem_in)
            pltpu.async_copy(mask_hbm.at[sl], mask2.at[b], sem_in)

        def drain_idx():
            for _ in range(4):
                pltpu.make_async_copy(pti_hbm.at[pl.ds(0, CHUNK)],
                                      pti2.at[0], sem_in).wait()
            pltpu.make_async_copy(mask_hbm.at[pl.ds(0, CHUNK)],
                                  mask2.at[0], sem_in).wait()

        def fire_heads(ci):
            sl = pl.ds(base0 + ci * CHUNK, CHUNK)
            pltpu.async_copy(ph_hbm.at[sl], ph_v, sem_heads)
            pltpu.async_copy(nh_hbm.at[sl], nh_v, sem_heads)

        def drain_heads():
            pltpu.make_async_copy(ph_hbm.at[pl.ds(0, CHUNK)],
                                  ph_v, sem_heads).wait()
            pltpu.make_async_copy(ph_hbm.at[pl.ds(0, CHUNK)],
                                  nh_v, sem_heads).wait()

        def fire_group(b, g):
            gsl = pl.ds(g * LANES, LANES)
            ptv = pti2[b, gsl]
            ntv = nti2[b, gsl]
            for rr in range(LANES):
                r = g * LANES + rr
                pltpu.async_copy(tail_hbm.at[pl.ds(ptv[rr], 1)],
                                 pt_v.at[b, pl.ds(r, 1)], sem_rows)
                pltpu.async_copy(tail_hbm.at[pl.ds(ntv[rr], 1)],
                                 nt_v.at[b, pl.ds(r, 1)], sem_rows)

        def drain_rows_all():
            def d(i, c):
                pltpu.make_async_copy(tail_hbm.at[pl.ds(0, 1)],
                                      pt_v.at[0, pl.ds(0, 1)],
                                      sem_rows).wait()
                return c

            lax.fori_loop(0, 2 * CHUNK, d, 0)

        def drain_outs():
            pltpu.make_async_copy(relout_hbm.at[pl.ds(0, CHUNK)],
                                  pr_v, sem_out).wait()

        def chunk_body(ci, carry):
            b = lax.rem(ci, 2)
            sl = pl.ds(base0 + ci * CHUNK, CHUNK)

            @pl.when(ci > 0)
            def _():
                drain_outs()

            drain_rows_all()

            @pl.when(ci + 1 < n_chunks)
            def _():
                drain_idx()

            drain_heads()

            def grand(gg, c):
                @pl.when(ci + 1 < n_chunks)
                def _():
                    fire_group(1 - b, gg)

                gsl = pl.ds(gg * LANES, LANES)
                prvec = pri2[b, gsl] * DIM
                nrvec = nri2[b, gsl] * DIM
                for rr in range(LANES):
                    r = gg * LANES + rr
                    pbase = prvec[rr]
                    nbase = nrvec[rr]
                    acc0 = jnp.zeros((LANES,), jnp.float32)
                    acc1 = jnp.zeros((LANES,), jnp.float32)
                    for j in range(DIM // LANES):
                        js = pl.ds(j * LANES, LANES)
                        prj = plsc.load_gather(
                            rel_v, [pbase + j * LANES + iota])
                        nrj = plsc.load_gather(
                            rel_v, [nbase + j * LANES + iota])
                        pr_v[r, js] = prj
                        pterm = jnp.abs(ph_v[r, js] + prj - pt_v[b, r, js])
                        nterm = jnp.abs(nh_v[r, js] + nrj - nt_v[b, r, js])
                        if j % 2 == 0:
                            acc0 = acc0 + (pterm - nterm)
                        else:
                            acc1 = acc1 + (pterm - nterm)
                    stage_v[pl.ds(rr * LANES, LANES)] = acc0 + acc1
                tot = jnp.zeros((LANES,), jnp.float32)
                for j in range(LANES):
                    tot = tot + plsc.load_gather(stage_v, [iota * LANES + j])
                loss_v[pl.ds(ci * CHUNK + gg * LANES, LANES)] = jnp.maximum(
                    mask2[b, gsl] * tot + MARGIN, 0.0)
                return c

            lax.fori_loop(0, n_groups, grand, 0)
            pltpu.async_copy(pr_v, relout_hbm.at[sl], sem_out)

            @pl.when(ci + 1 < n_chunks)
            def _():
                fire_heads(ci + 1)

            @pl.when(ci + 2 < n_chunks)
            def _():
                fire_idx(ci + 2, b)

            return carry

        pltpu.async_copy(relf_hbm, rel_v, sem_tbl)
        fire_idx(0, 0)
        pltpu.make_async_copy(relf_hbm, rel_v, sem_tbl).wait()
        drain_idx()

        def fire0(g, c):
            fire_group(0, g)
            return c

        lax.fori_loop(0, n_groups, fire0, 0)
        fire_idx(1, 1)
        fire_heads(0)
        lax.fori_loop(0, n_chunks, chunk_body, 0)
        drain_outs()
        pltpu.async_copy(loss_v, loss_hbm.at[pl.ds(base0, rows_per_w)],
                         sem_out)
        pltpu.make_async_copy(loss_v, loss_hbm.at[pl.ds(base0, rows_per_w)],
                              sem_out).wait()

    return k


def kernel(positive_head, positive_tail, positive_relation, negtive_head,
           negtive_tail, negtive_relation, attn_mask, rel_table, tail_table):
    b, l, d = positive_head.shape
    n = b * l
    ph = positive_head.reshape(n, d)
    nh = negtive_head.reshape(n, d)
    pti = positive_tail.reshape(n).astype(jnp.int32)
    nti = negtive_tail.reshape(n).astype(jnp.int32)
    pri = positive_relation.reshape(n).astype(jnp.int32)
    nri = negtive_relation.reshape(n).astype(jnp.int32)
    mask = attn_mask.reshape(n).astype(jnp.float32)
    rel_flat = rel_table.reshape(-1)
    loss, rel_rows = _build(n)(
        ph, nh, pti, nti, pri, nri, mask, rel_flat, tail_table)
    return loss.reshape(n, 1), rel_rows.reshape(b, l, d)


# speedup vs baseline: 1.3362x; 1.0216x over previous
import functools

import jax
import jax.numpy as jnp
from jax import lax
from jax.experimental import pallas as pl
from jax.experimental.pallas import tpu as pltpu
from jax.experimental.pallas import tpu_sc as plsc

DIM = 64
LANES = 16
CHUNK = 64
MARGIN = 1.0
REL_ROWS = 1001
REL_WORDS = REL_ROWS * DIM


def _build(n_rows):
    info = plsc.get_sparse_core_info()
    nc, ns = info.num_cores, info.num_subcores
    n_workers = nc * ns
    rows_per_w = n_rows // n_workers
    n_chunks = rows_per_w // CHUNK
    n_groups = CHUNK // LANES
    mesh = plsc.VectorSubcoreMesh(core_axis_name="c", subcore_axis_name="s")

    @functools.partial(
        pl.kernel,
        mesh=mesh,
        compiler_params=pltpu.CompilerParams(needs_layout_passes=False),
        out_type=(
            jax.ShapeDtypeStruct((n_rows,), jnp.float32),
            jax.ShapeDtypeStruct((n_rows, DIM), jnp.float32),
        ),
        scratch_types=[
            pltpu.VMEM((2, CHUNK), jnp.int32),
            pltpu.VMEM((2, CHUNK), jnp.int32),
            pltpu.VMEM((2, CHUNK), jnp.int32),
            pltpu.VMEM((2, CHUNK), jnp.int32),
            pltpu.VMEM((2, CHUNK), jnp.float32),
            pltpu.VMEM((CHUNK, DIM), jnp.float32),
            pltpu.VMEM((CHUNK, DIM), jnp.float32),
            pltpu.VMEM((2, CHUNK, DIM), jnp.float32),
            pltpu.VMEM((2, CHUNK, DIM), jnp.float32),
            pltpu.VMEM((CHUNK, DIM), jnp.float32),
            pltpu.VMEM((REL_WORDS,), jnp.float32),
            pltpu.VMEM((rows_per_w,), jnp.float32),
            pltpu.VMEM((LANES * LANES,), jnp.float32),
            pltpu.SemaphoreType.DMA,
            pltpu.SemaphoreType.DMA,
            pltpu.SemaphoreType.DMA,
            pltpu.SemaphoreType.DMA,
            pltpu.SemaphoreType.DMA,
        ],
    )
    def k(ph_hbm, nh_hbm, pti_hbm, nti_hbm, pri_hbm, nri_hbm, mask_hbm,
          relf_hbm, tail_hbm, loss_hbm, relout_hbm,
          pti2, nti2, pri2, nri2, mask2, ph_v, nh_v, pt_v, nt_v, pr_v,
          rel_v, loss_v, stage_v, sem_in, sem_heads, sem_rows, sem_out,
          sem_tbl):
        wid = lax.axis_index("s") * nc + lax.axis_index("c")
        base0 = wid * rows_per_w
        iota = lax.iota(jnp.int32, LANES)

        def fire_idx(ci, b):
            sl = pl.ds(base0 + ci * CHUNK, CHUNK)
            pltpu.async_copy(pti_hbm.at[sl], pti2.at[b], sem_in)
            pltpu.async_copy(nti_hbm.at[sl], nti2.at[b], sem_in)
            pltpu.async_copy(pri_hbm.at[sl], pri2.at[b], sem_in)
            pltpu.async_copy(nri_hbm.at[sl], nri2.at[b], sem_in)
            pltpu.async_copy(mask_hbm.at[sl], mask2.at[b], sem_in)

        def drain_idx():
            for _ in range(4):
                pltpu.make_async_copy(pti_hbm.at[pl.ds(0, CHUNK)],
                                      pti2.at[0], sem_in).wait()
            pltpu.make_async_copy(mask_hbm.at[pl.ds(0, CHUNK)],
                                  mask2.at[0], sem_in).wait()

        def fire_heads(ci):
            sl = pl.ds(base0 + ci * CHUNK, CHUNK)
            pltpu.async_copy(ph_hbm.at[sl], ph_v, sem_heads)
            pltpu.async_copy(nh_hbm.at[sl], nh_v, sem_heads)

        def drain_heads():
            pltpu.make_async_copy(ph_hbm.at[pl.ds(0, CHUNK)],
                                  ph_v, sem_heads).wait()
            pltpu.make_async_copy(ph_hbm.at[pl.ds(0, CHUNK)],
                                  nh_v, sem_heads).wait()

        def fire_group(b, g):
            gsl = pl.ds(g * LANES, LANES)
            ptv = pti2[b, gsl]
            ntv = nti2[b, gsl]
            for rr in range(LANES):
                r = g * LANES + rr
                pltpu.async_copy(tail_hbm.at[pl.ds(ptv[rr], 1)],
                                 pt_v.at[b, pl.ds(r, 1)], sem_rows)
                pltpu.async_copy(tail_hbm.at[pl.ds(ntv[rr], 1)],
                                 nt_v.at[b, pl.ds(r, 1)], sem_rows)

        def drain_rows_all():
            def d(i, c):
                pltpu.make_async_copy(tail_hbm.at[pl.ds(0, 1)],
                                      pt_v.at[0, pl.ds(0, 1)],
                                      sem_rows).wait()
                return c

            lax.fori_loop(0, 2 * CHUNK, d, 0)

        def drain_outs():
            pltpu.make_async_copy(relout_hbm.at[pl.ds(0, CHUNK)],
                                  pr_v, sem_out).wait()

        def chunk_body(ci, carry):
            b = lax.rem(ci, 2)
            sl = pl.ds(base0 + ci * CHUNK, CHUNK)

            @pl.when(ci > 0)
            def _():
                drain_outs()

            drain_rows_all()

            @pl.when(ci + 1 < n_chunks)
            def _():
                drain_idx()

            @pl.when(ci + 1 < n_chunks)
            def _():
                def firenext(g, c):
                    fire_group(1 - b, g)
                    return c

                lax.fori_loop(0, n_groups, firenext, 0)

            drain_heads()

            def grand(gg, c):
                gsl = pl.ds(gg * LANES, LANES)
                prvec = pri2[b, gsl] * DIM
                nrvec = nri2[b, gsl] * DIM
                for rr in range(LANES):
                    r = gg * LANES + rr
                    pbase = prvec[rr]
                    nbase = nrvec[rr]
                    acc0 = jnp.zeros((LANES,), jnp.float32)
                    acc1 = jnp.zeros((LANES,), jnp.float32)
                    for j in range(DIM // LANES):
                        js = pl.ds(j * LANES, LANES)
                        prj = plsc.load_gather(
                            rel_v, [pbase + j * LANES + iota])
                        nrj = plsc.load_gather(
                            rel_v, [nbase + j * LANES + iota])
                        pr_v[r, js] = prj
                        pterm = jnp.abs(ph_v[r, js] + prj - pt_v[b, r, js])
                        nterm = jnp.abs(nh_v[r, js] + nrj - nt_v[b, r, js])
                        if j % 2 == 0:
                            acc0 = acc0 + (pterm - nterm)
                        else:
                            acc1 = acc1 + (pterm - nterm)
                    stage_v[pl.ds(rr * LANES, LANES)] = acc0 + acc1
                tot = jnp.zeros((LANES,), jnp.float32)
                for j in range(LANES):
                    tot = tot + plsc.load_gather(stage_v, [iota * LANES + j])
                loss_v[pl.ds(ci * CHUNK + gg * LANES, LANES)] = jnp.maximum(
                    mask2[b, gsl] * tot + MARGIN, 0.0)
                return c

            lax.fori_loop(0, n_groups, grand, 0)
            pltpu.async_copy(pr_v, relout_hbm.at[sl], sem_out)

            @pl.when(ci + 1 < n_chunks)
            def _():
                fire_heads(ci + 1)

            @pl.when(ci + 2 < n_chunks)
            def _():
                fire_idx(ci + 2, b)

            return carry

        pltpu.async_copy(relf_hbm, rel_v, sem_tbl)
        fire_idx(0, 0)
        pltpu.make_async_copy(relf_hbm, rel_v, sem_tbl).wait()
        drain_idx()

        def fire0(g, c):
            fire_group(0, g)
            return c

        lax.fori_loop(0, n_groups, fire0, 0)
        fire_idx(1, 1)
        fire_heads(0)
        lax.fori_loop(0, n_chunks, chunk_body, 0)
        drain_outs()
        pltpu.async_copy(loss_v, loss_hbm.at[pl.ds(base0, rows_per_w)],
                         sem_out)
        pltpu.make_async_copy(loss_v, loss_hbm.at[pl.ds(base0, rows_per_w)],
                              sem_out).wait()

    return k


def kernel(positive_head, positive_tail, positive_relation, negtive_head,
           negtive_tail, negtive_relation, attn_mask, rel_table, tail_table):
    b, l, d = positive_head.shape
    n = b * l
    ph = positive_head.reshape(n, d)
    nh = negtive_head.reshape(n, d)
    pti = positive_tail.reshape(n).astype(jnp.int32)
    nti = negtive_tail.reshape(n).astype(jnp.int32)
    pri = positive_relation.reshape(n).astype(jnp.int32)
    nri = negtive_relation.reshape(n).astype(jnp.int32)
    mask = attn_mask.reshape(n).astype(jnp.float32)
    rel_flat = rel_table.reshape(-1)
    loss, rel_rows = _build(n)(
        ph, nh, pti, nti, pri, nri, mask, rel_flat, tail_table)
    return loss.reshape(n, 1), rel_rows.reshape(b, l, d)
